# baseline (speedup 1.0000x reference)
import jax
import jax.numpy as jnp
from jax import lax
from jax.experimental import pallas as pl
from jax.experimental.pallas import tpu as pltpu
from jax.experimental.pallas import tpu_sc as plsc

B = 16384
CTX = 20
NOISE = 5
VD = 64
NC = 2
NS = 16
NW = NC * NS
BPW = B // NW
CB = 128
NCHUNK = BPW // CB
NPAIR = CB * NOISE
NGRP = NPAIR // 16


def _sc_body(ctxt_ref, doc_ref, noise_ref, d_tab, w_tab, ot_tab, out_ref,
             ctxt_idx, noise_idx, doc_idx, x_bufs, g_bufs, out_buf,
             dsems, wsems, gsems):
    wid = lax.axis_index("s") * NC + lax.axis_index("c")
    lane = lax.iota(jnp.int32, 16)

    for c in range(CTX):
        pltpu.sync_copy(ctxt_ref.at[pl.ds(c * B + wid * BPW, BPW)],
                        ctxt_idx.at[pl.ds(c * BPW, BPW)])
    pltpu.sync_copy(noise_ref.at[pl.ds(wid * BPW * NOISE, BPW * NOISE)],
                    noise_idx)
    pltpu.sync_copy(doc_ref.at[pl.ds(wid * BPW, BPW)], doc_idx)

    def d_copy(kk, slot):
        return pltpu.make_async_copy(
            d_tab.at[doc_idx.at[pl.ds(kk * CB, CB)]], x_bufs[slot],
            dsems[slot])

    def w_src(kk, c):
        return w_tab.at[ctxt_idx.at[pl.ds(c * BPW + kk * CB, CB)]]

    def g_copy(kk, slot):
        return pltpu.make_async_copy(
            ot_tab.at[noise_idx.at[pl.ds(kk * CB * NOISE, CB * NOISE)]],
            g_bufs[slot], gsems[slot])

    def issue_adds(kk, slot):
        d_copy(kk, slot).wait()
        for c in range(CTX):
            pltpu.async_copy(w_src(kk, c), x_bufs[slot], wsems[slot],
                             add=True)
        g_copy(kk, slot).start()

    def drain_adds(kk, slot):
        for c in range(CTX):
            pltpu.make_async_copy(w_src(kk, c), x_bufs[slot],
                                  wsems[slot]).wait()
        g_copy(kk, slot).wait()

    def compute(kk, slot):
        x_buf, g_buf = x_bufs[slot], g_bufs[slot]

        @pl.loop(0, NGRP)
        def _grp(g):
            g16 = lax.broadcast(g * 16, (16,))
            p = g16 + lane
            b_v = lax.div(p, jnp.full((16,), NOISE, jnp.int32))
            accs = [jnp.zeros((16,), jnp.float32) for _ in range(4)]
            for d in range(VD):
                cd = jnp.full((16,), d, jnp.int32)
                xv = plsc.load_gather(x_buf, [b_v, cd])
                gv = plsc.load_gather(g_buf, [p, cd])
                accs[d % 4] = accs[d % 4] + xv * gv
            acc = (accs[0] + accs[1]) + (accs[2] + accs[3])
            out_buf[pl.ds(g * 16, 16)] = acc

        pltpu.sync_copy(out_buf,
                        out_ref.at[pl.ds(wid * BPW * NOISE + kk * NPAIR,
                                         NPAIR)])

    d_copy(0, 0).start()
    issue_adds(0, 0)
    d_copy(1, 1).start()
    for k in range(NCHUNK):
        s = k % 2
        o = (k + 1) % 2
        drain_adds(k, s)
        if k + 1 < NCHUNK:
            issue_adds(k + 1, o)
        compute(k, s)
        if k + 2 < NCHUNK:
            d_copy(k + 2, s).start()


@jax.jit
def _dm_forward(ctxt_flat, doc_ids, noise_flat, D, W, OT):
    mesh = plsc.VectorSubcoreMesh(core_axis_name="c", subcore_axis_name="s",
                                  num_cores=NC, num_subcores=NS)
    f = pl.kernel(
        _sc_body,
        out_type=jax.ShapeDtypeStruct((B * NOISE,), jnp.float32),
        mesh=mesh,
        scratch_types=[
            pltpu.VMEM((BPW * CTX,), jnp.int32),
            pltpu.VMEM((BPW * NOISE,), jnp.int32),
            pltpu.VMEM((BPW,), jnp.int32),
            [pltpu.VMEM((CB, VD), jnp.float32) for _ in range(2)],
            [pltpu.VMEM((NPAIR, VD), jnp.float32) for _ in range(2)],
            pltpu.VMEM((NPAIR,), jnp.float32),
            [pltpu.SemaphoreType.DMA for _ in range(2)],
            [pltpu.SemaphoreType.DMA for _ in range(2)],
            [pltpu.SemaphoreType.DMA for _ in range(2)],
        ],
        compiler_params=pltpu.CompilerParams(use_tc_tiling_on_sc=False,
                                             needs_layout_passes=False),
    )
    return f(ctxt_flat, doc_ids, noise_flat, D, W, OT)


def kernel(context_ids, doc_ids, target_noise_ids, D, W, O):
    ctxt = context_ids.T.reshape(-1)
    OT = O.T
    out = _dm_forward(ctxt, doc_ids, target_noise_ids.reshape(-1), D, W, OT)
    return out.reshape(B, NOISE)


# speedup vs baseline: 1.4400x; 1.4400x over previous
import jax
import jax.numpy as jnp
from jax import lax
from jax.experimental import pallas as pl
from jax.experimental.pallas import tpu as pltpu
from jax.experimental.pallas import tpu_sc as plsc

B = 16384
CTX = 20
NOISE = 5
VD = 64
NC = 2
NS = 16
NW = NC * NS
BPW = B // NW
CB = 128
NCHUNK = BPW // CB
NPAIR = CB * NOISE
NGRP = NPAIR // 16


def _sc_body(ctxt_ref, doc_ref, noise_ref, d_tab, w_tab, ot_tab, out_ref,
             ctxt_idx, noise_idx, doc_idx, x_bufs, g_bufs, out_buf,
             dsems, wsems, gsems):
    wid = lax.axis_index("s") * NC + lax.axis_index("c")
    lane = lax.iota(jnp.int32, 16)

    for c in range(CTX):
        pltpu.sync_copy(ctxt_ref.at[pl.ds(c * B + wid * BPW, BPW)],
                        ctxt_idx.at[pl.ds(c * BPW, BPW)])
    pltpu.sync_copy(noise_ref.at[pl.ds(wid * BPW * NOISE, BPW * NOISE)],
                    noise_idx)
    pltpu.sync_copy(doc_ref.at[pl.ds(wid * BPW, BPW)], doc_idx)

    def d_copy(kk, slot):
        return pltpu.make_async_copy(
            d_tab.at[doc_idx.at[pl.ds(kk * CB, CB)]], x_bufs[slot],
            dsems[slot])

    def w_src(kk, c):
        return w_tab.at[ctxt_idx.at[pl.ds(c * BPW + kk * CB, CB)]]

    def g_copy(kk, slot):
        return pltpu.make_async_copy(
            ot_tab.at[noise_idx.at[pl.ds(kk * CB * NOISE, CB * NOISE)]],
            g_bufs[slot], gsems[slot])

    def issue_adds(kk, slot):
        d_copy(kk, slot).wait()
        for c in range(CTX):
            pltpu.async_copy(w_src(kk, c), x_bufs[slot], wsems[slot],
                             add=True)
        g_copy(kk, slot).start()

    def drain_adds(kk, slot):
        for c in range(CTX):
            pltpu.make_async_copy(w_src(kk, c), x_bufs[slot],
                                  wsems[slot]).wait()
        g_copy(kk, slot).wait()

    def compute(kk, slot):
        x_buf, g_buf = x_bufs[slot], g_bufs[slot]

        perms = [lane ^ jnp.full((16,), kk, jnp.int32) for kk in (8, 4, 2, 1)]

        @pl.loop(0, NGRP)
        def _pg(g):
            out_vec = jnp.zeros((16,), jnp.float32)
            for j in range(16):
                p = g * 16 + j
                b = lax.div(p, NOISE)
                s = x_buf[b, pl.ds(0, 16)] * g_buf[p, pl.ds(0, 16)]
                for q in range(1, 4):
                    s = s + (x_buf[b, pl.ds(q * 16, 16)]
                             * g_buf[p, pl.ds(q * 16, 16)])
                for pm in perms:
                    s = s + jax.lax.gather(
                        s, pm[:, None],
                        jax.lax.GatherDimensionNumbers(
                            offset_dims=(), collapsed_slice_dims=(0,),
                            start_index_map=(0,)),
                        (1,), mode=jax.lax.GatherScatterMode.PROMISE_IN_BOUNDS)
                out_vec = jnp.where(lane == jnp.full((16,), j, jnp.int32),
                                    s, out_vec)
            out_buf[pl.ds(g * 16, 16)] = out_vec

        pltpu.sync_copy(out_buf,
                        out_ref.at[pl.ds(wid * BPW * NOISE + kk * NPAIR,
                                         NPAIR)])

    d_copy(0, 0).start()
    issue_adds(0, 0)
    d_copy(1, 1).start()
    for k in range(NCHUNK):
        s = k % 2
        o = (k + 1) % 2
        drain_adds(k, s)
        if k + 1 < NCHUNK:
            issue_adds(k + 1, o)
        compute(k, s)
        if k + 2 < NCHUNK:
            d_copy(k + 2, s).start()


@jax.jit
def _dm_forward(ctxt_flat, doc_ids, noise_flat, D, W, OT):
    mesh = plsc.VectorSubcoreMesh(core_axis_name="c", subcore_axis_name="s",
                                  num_cores=NC, num_subcores=NS)
    f = pl.kernel(
        _sc_body,
        out_type=jax.ShapeDtypeStruct((B * NOISE,), jnp.float32),
        mesh=mesh,
        scratch_types=[
            pltpu.VMEM((BPW * CTX,), jnp.int32),
            pltpu.VMEM((BPW * NOISE,), jnp.int32),
            pltpu.VMEM((BPW,), jnp.int32),
            [pltpu.VMEM((CB, VD), jnp.float32) for _ in range(2)],
            [pltpu.VMEM((NPAIR, VD), jnp.float32) for _ in range(2)],
            pltpu.VMEM((NPAIR,), jnp.float32),
            [pltpu.SemaphoreType.DMA for _ in range(2)],
            [pltpu.SemaphoreType.DMA for _ in range(2)],
            [pltpu.SemaphoreType.DMA for _ in range(2)],
        ],
        compiler_params=pltpu.CompilerParams(use_tc_tiling_on_sc=False,
                                             needs_layout_passes=False),
    )
    return f(ctxt_flat, doc_ids, noise_flat, D, W, OT)


def kernel(context_ids, doc_ids, target_noise_ids, D, W, O):
    ctxt = context_ids.T.reshape(-1)
    OT = O.T
    out = _dm_forward(ctxt, doc_ids, target_noise_ids.reshape(-1), D, W, OT)
    return out.reshape(B, NOISE)
